# baseline (speedup 1.0000x reference)
import functools

import jax
import jax.numpy as jnp
from jax import lax
from jax.experimental import pallas as pl
from jax.experimental.pallas import tpu as pltpu
from jax.experimental.pallas import tpu_sc as plsc

N = 10000
F = 128
FO = 64
E = 320000
E_SUB = 20480
E_PAD = E_SUB * 16
CH = 128
NCH = E_SUB // CH
NROW = N + 16
SLAB = 632
ZTAIL = NROW - 15 * SLAB
OTAIL = N - 15 * SLAB
CW = 16

E_LBL = 100000
P_LBL = 102400
P_SUB = P_LBL // 32
P_NCH = P_SUB // CH

_MESH = plsc.VectorSubcoreMesh(core_axis_name="c", subcore_axis_name="s")
_SC_PARAMS = pltpu.CompilerParams(use_tc_tiling_on_sc=False)


def _f32(*shape):
    return jax.ShapeDtypeStruct(shape, jnp.float32)


NB = 4
NB1 = 2


def _make_agg_kernel(width, with_counts, nb):
    scratch = [
        pltpu.VMEM((nb, CH), jnp.int32),
        pltpu.VMEM((2 * nb, CH), jnp.int32),
        pltpu.VMEM((nb, CH, width), jnp.float32),
    ] + [pltpu.SemaphoreType.DMA] * (3 * nb) + [
        pltpu.VMEM_SHARED((NROW, width), jnp.float32),
    ]
    out_type = [_f32(N, width), _f32(N, width)]
    if with_counts:
        scratch.append(pltpu.VMEM((CH, CW), jnp.float32))
        scratch.append(pltpu.VMEM_SHARED((NROW, CW), jnp.float32))
        out_type += [_f32(N, CW), _f32(N, CW)]

    @functools.partial(pl.kernel, mesh=_MESH, out_type=out_type,
                       scratch_types=scratch, compiler_params=_SC_PARAMS)
    def agg(tab_u, tab_i, su, du, si, di, zrow, zcnt, ones_hbm, *rest):
        n_out = 4 if with_counts else 2
        if with_counts:
            agg_i, agg_u, cnt_i, cnt_u = rest[:4]
        else:
            agg_i, agg_u = rest[:2]
            cnt_i = cnt_u = None
        sc = rest[n_out:]
        idx_s, idx_d, rows = sc[:3]
        sem_i = sc[3:3 + nb]
        sem_g = sc[3 + nb:3 + 2 * nb]
        sem_s = sc[3 + 2 * nb:3 + 3 * nb]
        acc_sh = sc[3 + 3 * nb]
        if with_counts:
            ones_v, cnt_sh = sc[4 + 3 * nb:]
        c = lax.axis_index("c")
        s = lax.axis_index("s")

        def zero_main():
            pltpu.sync_copy(zrow, acc_sh.at[pl.ds(s * SLAB, SLAB)])
            if with_counts:
                pltpu.sync_copy(zcnt, cnt_sh.at[pl.ds(s * SLAB, SLAB)])

        def zero_tail():
            pltpu.sync_copy(zrow.at[pl.ds(0, ZTAIL)],
                            acc_sh.at[pl.ds(15 * SLAB, ZTAIL)])
            if with_counts:
                pltpu.sync_copy(zcnt.at[pl.ds(0, ZTAIL)],
                                cnt_sh.at[pl.ds(15 * SLAB, ZTAIL)])

        pl.when(s < 15)(zero_main)
        pl.when(s == 15)(zero_tail)
        if with_counts:
            pltpu.sync_copy(ones_hbm, ones_v)
        plsc.subcore_barrier()

        def run_edges(src_h, dst_h, tab_h):
            base0 = s * E_SUB
            ngrp = NCH // nb

            def start_idx(b, chunk, bank):
                pltpu.async_copy(src_h.at[pl.ds(base0 + chunk * CH, CH)],
                                 idx_s.at[b], sem_i[b])
                pltpu.async_copy(dst_h.at[pl.ds(base0 + chunk * CH, CH)],
                                 idx_d.at[bank * nb + b], sem_i[b])

            def wait_idx(b):
                pltpu.make_async_copy(src_h.at[pl.ds(base0, CH)],
                                      idx_s.at[b], sem_i[b]).wait()
                pltpu.make_async_copy(src_h.at[pl.ds(base0, CH)],
                                      idx_s.at[b], sem_i[b]).wait()

            def wait_scatter(b, bank):
                pltpu.make_async_copy(rows.at[b],
                                      acc_sh.at[idx_d.at[bank * nb + b]],
                                      sem_s[b]).wait()

            for b in range(nb):
                start_idx(b, b, 0)

            def phase(t, grp, bank, first):
                handles = []
                for b in range(nb):
                    wait_idx(b)
                    if first:
                        pl.when(t > 0)(
                            lambda b=b: wait_scatter(b, 1 - bank))
                    else:
                        wait_scatter(b, 1 - bank)
                    handles.append(pltpu.async_copy(
                        tab_h.at[idx_s.at[b]], rows.at[b], sem_g[b]))
                for b in range(nb):
                    handles[b].wait()
                    pltpu.async_copy(rows.at[b],
                                     acc_sh.at[idx_d.at[bank * nb + b]],
                                     sem_s[b], add=True)
                    start_idx(b, jnp.minimum((grp + 1) * nb + b, NCH - 1),
                              1 - bank)

            def step(t, carry):
                phase(t, t * 2, 0, True)
                phase(t, t * 2 + 1, 1, False)
                return carry
            lax.fori_loop(0, ngrp // 2, step, 0)
            last_bank = 1
            for b in range(nb):
                wait_scatter(b, last_bank)
                wait_idx(b)

        pl.when(c == 0)(lambda: run_edges(su, du, tab_u))
        pl.when(c == 1)(lambda: run_edges(si, di, tab_i))
        plsc.subcore_barrier()

        def copy_out(agg_out, cnt_out):
            def out_main():
                pltpu.sync_copy(acc_sh.at[pl.ds(s * SLAB, SLAB)],
                                agg_out.at[pl.ds(s * SLAB, SLAB)])
                if with_counts:
                    pltpu.sync_copy(cnt_sh.at[pl.ds(s * SLAB, SLAB)],
                                    cnt_out.at[pl.ds(s * SLAB, SLAB)])

            def out_tail():
                pltpu.sync_copy(acc_sh.at[pl.ds(15 * SLAB, OTAIL)],
                                agg_out.at[pl.ds(15 * SLAB, OTAIL)])
                if with_counts:
                    pltpu.sync_copy(cnt_sh.at[pl.ds(15 * SLAB, OTAIL)],
                                    cnt_out.at[pl.ds(15 * SLAB, OTAIL)])

            pl.when(s < 15)(out_main)
            pl.when(s == 15)(out_tail)

        if with_counts:
            pl.when(c == 0)(lambda: copy_out(agg_i, cnt_i))
            pl.when(c == 1)(lambda: copy_out(agg_u, cnt_u))
        else:
            pl.when(c == 0)(lambda: copy_out(agg_i, None))
            pl.when(c == 1)(lambda: copy_out(agg_u, None))

    return agg


_agg_l1 = _make_agg_kernel(F, True, NB1)
_agg_l2 = _make_agg_kernel(FO, False, NB)


@functools.partial(
    pl.kernel, mesh=_MESH, out_type=[_f32(P_LBL, FO), _f32(P_LBL, FO)],
    compiler_params=_SC_PARAMS,
    scratch_types=[
        pltpu.VMEM((NB, CH), jnp.int32),
        pltpu.VMEM((NB, CH), jnp.int32),
        pltpu.VMEM((NB, CH, FO), jnp.float32),
        pltpu.VMEM((NB, CH, FO), jnp.float32),
    ] + [pltpu.SemaphoreType.DMA] * (2 * NB))
def _pair_gather(zu, zi, l0, l1, f1_out, f2_out, i1, i2, f1, f2, *sems):
    sem_i = sems[:NB]
    sem_g = sems[NB:]
    c = lax.axis_index("c")
    s = lax.axis_index("s")
    wid = s * 2 + c
    base0 = wid * P_SUB

    def start_idx(b, chunk):
        pltpu.async_copy(l0.at[pl.ds(base0 + chunk * CH, CH)],
                         i1.at[b], sem_i[b])
        pltpu.async_copy(l1.at[pl.ds(base0 + chunk * CH, CH)],
                         i2.at[b], sem_i[b])

    def wait_idx(b):
        pltpu.make_async_copy(l0.at[pl.ds(base0, CH)], i1.at[b],
                              sem_i[b]).wait()
        pltpu.make_async_copy(l1.at[pl.ds(base0, CH)], i2.at[b],
                              sem_i[b]).wait()

    for b in range(NB):
        start_idx(b, b)

    def step(t, carry):
        g = t * NB
        handles = []
        for b in range(NB):
            wait_idx(b)
            handles.append(pltpu.async_copy(zu.at[i1.at[b]], f1.at[b],
                                            sem_g[b]))
            handles.append(pltpu.async_copy(zi.at[i2.at[b]], f2.at[b],
                                            sem_g[b]))
        for b in range(NB):
            base = base0 + jnp.minimum(g + b, P_NCH - 1) * CH
            handles[2 * b].wait()
            handles[2 * b + 1].wait()
            pltpu.sync_copy(f1.at[b], f1_out.at[pl.ds(base, CH)])
            pltpu.sync_copy(f2.at[b], f2_out.at[pl.ds(base, CH)])
            start_idx(b, jnp.minimum(g + NB + b, P_NCH - 1))
        return carry
    lax.fori_loop(0, (P_NCH + NB - 1) // NB, step, 0)
    for b in range(NB):
        wait_idx(b)


_DR = 10240


def _dot_body(f1_ref, f2_ref, o_ref):
    o_ref[...] = jnp.sum(f1_ref[...] * f2_ref[...], axis=1).reshape(
        _DR // 128, 128)


def _pair_dot(f1, f2):
    return pl.pallas_call(
        _dot_body,
        grid=(P_LBL // _DR,),
        in_specs=[pl.BlockSpec((_DR, FO), lambda i: (i, 0)),
                  pl.BlockSpec((_DR, FO), lambda i: (i, 0))],
        out_specs=pl.BlockSpec((_DR // 128, 128), lambda i: (i, 0)),
        out_shape=_f32(P_LBL // 128, 128),
    )(f1, f2)


def _dotT(a, b):
    return lax.dot_general(a, b, (((1,), (1,)), ((), ())),
                           preferred_element_type=jnp.float32)


_BR = 1000


def _mlp_one(agg_ref, cnt_ref, x_ref, w1l_ref, b1_ref, w1r_ref,
             w2a_ref, w2b_ref, b2_ref, p_ref, s_ref):
    cnt = jnp.maximum(cnt_ref[:, 0:1], 1.0)
    mean = agg_ref[...] / cnt
    h = jnp.maximum(
        _dotT(mean, w1l_ref[...]) + b1_ref[...] + _dotT(x_ref[...], w1r_ref[...]),
        0.0)
    p_ref[...] = _dotT(h, w2a_ref[...])
    s_ref[...] = _dotT(h, w2b_ref[...]) + b2_ref[...]


def _mlp2_body(*refs):
    _mlp_one(*refs[0:9], *refs[18:20])
    _mlp_one(*refs[9:18], *refs[20:22])


def _mlp2(argsi, argsu):
    full = lambda shp: pl.BlockSpec(shp, lambda i: (0, 0))
    row = lambda w: pl.BlockSpec((_BR, w), lambda i: (i, 0))
    per_type = [row(F), row(CW), row(F), full((F, F)), full((1, F)),
                full((F, F)), full((FO, F)), full((FO, F)), full((1, FO))]

    def prep(a):
        agg, cnt, x, w1l, b1, w1r, w2a, w2b, b2 = a
        return (agg, cnt, x, w1l, b1.reshape(1, F), w1r, w2a, w2b,
                b2.reshape(1, FO))

    return pl.pallas_call(
        _mlp2_body,
        grid=(N // _BR,),
        in_specs=per_type + per_type,
        out_specs=[row(FO)] * 4,
        out_shape=[_f32(N, FO)] * 4,
    )(*prep(argsi), *prep(argsu))


def _finish2_body(agg_i_ref, cnt_i_ref, s_i_ref, agg_u_ref, cnt_u_ref,
                  s_u_ref, z_i_ref, z_u_ref):
    z_i_ref[...] = agg_i_ref[...] / jnp.maximum(cnt_i_ref[:, 0:1], 1.0) + s_i_ref[...]
    z_u_ref[...] = agg_u_ref[...] / jnp.maximum(cnt_u_ref[:, 0:1], 1.0) + s_u_ref[...]


def _finish2(agg2_i, cnt_i, s_i, agg2_u, cnt_u, s_u):
    row = lambda w: pl.BlockSpec((_BR, w), lambda i: (i, 0))
    return pl.pallas_call(
        _finish2_body,
        grid=(N // _BR,),
        in_specs=[row(FO), row(CW), row(FO)] * 2,
        out_specs=[row(FO)] * 2,
        out_shape=[_f32(N, FO)] * 2,
    )(agg2_i, cnt_i, s_i, agg2_u, cnt_u, s_u)


def kernel(x_user, x_item, edge_index_u2i, edge_index_i2u, edge_label_index,
           W1_l_u2i, b1_u2i, W1_r_u2i, W1_l_i2u, b1_i2u, W1_r_i2u,
           W2_l_u2i, b2_u2i, W2_r_u2i, W2_l_i2u, b2_i2u, W2_r_i2u):
    i32 = jnp.int32
    pad_e = E_PAD - E
    su = jnp.concatenate([edge_index_u2i[0].astype(i32), jnp.zeros((pad_e,), i32)])
    du = jnp.concatenate([edge_index_u2i[1].astype(i32), jnp.full((pad_e,), N, i32)])
    si = jnp.concatenate([edge_index_i2u[0].astype(i32), jnp.zeros((pad_e,), i32)])
    di = jnp.concatenate([edge_index_i2u[1].astype(i32), jnp.full((pad_e,), N, i32)])
    pad_l = P_LBL - edge_label_index.shape[1]
    l0 = jnp.concatenate([edge_label_index[0].astype(i32), jnp.zeros((pad_l,), i32)])
    l1 = jnp.concatenate([edge_label_index[1].astype(i32), jnp.zeros((pad_l,), i32)])

    zrow = jnp.zeros((SLAB, F), jnp.float32)
    zcnt = jnp.zeros((SLAB, CW), jnp.float32)
    ones = jnp.ones((CH, CW), jnp.float32)

    agg_i, agg_u, cnt_i, cnt_u = _agg_l1(
        x_user, x_item, su, du, si, di, zrow, zcnt, ones)

    p_item, s_item, p_user, s_user = _mlp2(
        (agg_i, cnt_i, x_item, W1_l_u2i, b1_u2i, W1_r_u2i,
         W2_l_i2u, W2_r_u2i, b2_u2i),
        (agg_u, cnt_u, x_user, W1_l_i2u, b1_i2u, W1_r_i2u,
         W2_l_u2i, W2_r_i2u, b2_i2u))

    zrow2 = jnp.zeros((SLAB, FO), jnp.float32)
    agg2_i, agg2_u = _agg_l2(p_user, p_item, su, du, si, di, zrow2, zcnt, ones)

    z_item, z_user = _finish2(agg2_i, cnt_i, s_item, agg2_u, cnt_u, s_user)

    f1, f2 = _pair_gather(z_user, z_item, l0, l1)
    pred = _pair_dot(f1, f2)
    return pred.reshape(P_LBL)[:E_LBL]


# speedup vs baseline: 1.0810x; 1.0810x over previous
import functools

import jax
import jax.numpy as jnp
from jax import lax
from jax.experimental import pallas as pl
from jax.experimental.pallas import tpu as pltpu
from jax.experimental.pallas import tpu_sc as plsc

N = 10000
F = 128
FO = 64
E = 320000
E_SUB = 20480
E_PAD = E_SUB * 16
CH = 128
NCH = E_SUB // CH
NROW = N + 16
SLAB = 632
ZTAIL = NROW - 15 * SLAB
OTAIL = N - 15 * SLAB
CW = 16

E_LBL = 100000
P_LBL = 102400
P_SUB = P_LBL // 32
P_NCH = P_SUB // CH

_MESH = plsc.VectorSubcoreMesh(core_axis_name="c", subcore_axis_name="s")
_SC_PARAMS = pltpu.CompilerParams(use_tc_tiling_on_sc=False)


def _f32(*shape):
    return jax.ShapeDtypeStruct(shape, jnp.float32)


NB = 4
NB1 = 2


def _make_agg_kernel(width, with_counts, nb, stage_tab=False):
    scratch = [
        pltpu.VMEM((nb, CH), jnp.int32),
        pltpu.VMEM((2 * nb, CH), jnp.int32),
        pltpu.VMEM((nb, CH, width), jnp.float32),
    ] + [pltpu.SemaphoreType.DMA] * (3 * nb) + [
        pltpu.VMEM_SHARED((NROW, width), jnp.float32),
    ]
    if stage_tab:
        scratch.append(pltpu.VMEM_SHARED((N, width), jnp.float32))
    out_type = [_f32(N, width), _f32(N, width)]
    if with_counts:
        scratch.append(pltpu.VMEM((CH, CW), jnp.float32))
        scratch.append(pltpu.VMEM_SHARED((NROW, CW), jnp.float32))
        out_type += [_f32(N, CW), _f32(N, CW)]

    @functools.partial(pl.kernel, mesh=_MESH, out_type=out_type,
                       scratch_types=scratch, compiler_params=_SC_PARAMS)
    def agg(tab_u, tab_i, su, du, si, di, zrow, zcnt, ones_hbm, *rest):
        n_out = 4 if with_counts else 2
        if with_counts:
            agg_i, agg_u, cnt_i, cnt_u = rest[:4]
        else:
            agg_i, agg_u = rest[:2]
            cnt_i = cnt_u = None
        sc = rest[n_out:]
        idx_s, idx_d, rows = sc[:3]
        sem_i = sc[3:3 + nb]
        sem_g = sc[3 + nb:3 + 2 * nb]
        sem_s = sc[3 + 2 * nb:3 + 3 * nb]
        acc_sh = sc[3 + 3 * nb]
        off = 4 + 3 * nb
        if stage_tab:
            tab_sh = sc[off]
            off += 1
        if with_counts:
            ones_v, cnt_sh = sc[off:]
        c = lax.axis_index("c")
        s = lax.axis_index("s")

        def zero_main():
            pltpu.sync_copy(zrow, acc_sh.at[pl.ds(s * SLAB, SLAB)])
            if with_counts:
                pltpu.sync_copy(zcnt, cnt_sh.at[pl.ds(s * SLAB, SLAB)])

        def zero_tail():
            pltpu.sync_copy(zrow.at[pl.ds(0, ZTAIL)],
                            acc_sh.at[pl.ds(15 * SLAB, ZTAIL)])
            if with_counts:
                pltpu.sync_copy(zcnt.at[pl.ds(0, ZTAIL)],
                                cnt_sh.at[pl.ds(15 * SLAB, ZTAIL)])

        pl.when(s < 15)(zero_main)
        pl.when(s == 15)(zero_tail)
        if stage_tab:
            def stage(tab_h):
                pl.when(s < 15)(lambda: pltpu.sync_copy(
                    tab_h.at[pl.ds(s * SLAB, SLAB)],
                    tab_sh.at[pl.ds(s * SLAB, SLAB)]))
                pl.when(s == 15)(lambda: pltpu.sync_copy(
                    tab_h.at[pl.ds(15 * SLAB, OTAIL)],
                    tab_sh.at[pl.ds(15 * SLAB, OTAIL)]))
            pl.when(c == 0)(lambda: stage(tab_u))
            pl.when(c == 1)(lambda: stage(tab_i))
        if with_counts:
            pltpu.sync_copy(ones_hbm, ones_v)
        plsc.subcore_barrier()

        def run_edges(src_h, dst_h, tab_h):
            base0 = s * E_SUB
            ngrp = NCH // nb

            def start_idx(b, chunk, bank):
                pltpu.async_copy(src_h.at[pl.ds(base0 + chunk * CH, CH)],
                                 idx_s.at[b], sem_i[b])
                pltpu.async_copy(dst_h.at[pl.ds(base0 + chunk * CH, CH)],
                                 idx_d.at[bank * nb + b], sem_i[b])

            def wait_idx(b):
                pltpu.make_async_copy(src_h.at[pl.ds(base0, CH)],
                                      idx_s.at[b], sem_i[b]).wait()
                pltpu.make_async_copy(src_h.at[pl.ds(base0, CH)],
                                      idx_s.at[b], sem_i[b]).wait()

            def wait_scatter(b, bank):
                pltpu.make_async_copy(rows.at[b],
                                      acc_sh.at[idx_d.at[bank * nb + b]],
                                      sem_s[b]).wait()

            for b in range(nb):
                start_idx(b, b, 0)

            def phase(t, grp, bank, first):
                handles = []
                for b in range(nb):
                    wait_idx(b)
                    if first:
                        pl.when(t > 0)(
                            lambda b=b: wait_scatter(b, 1 - bank))
                    else:
                        wait_scatter(b, 1 - bank)
                    handles.append(pltpu.async_copy(
                        tab_h.at[idx_s.at[b]], rows.at[b], sem_g[b]))
                    if with_counts:
                        pltpu.sync_copy(ones_v,
                                        cnt_sh.at[idx_d.at[bank * nb + b]],
                                        add=True)
                for b in range(nb):
                    handles[b].wait()
                    pltpu.async_copy(rows.at[b],
                                     acc_sh.at[idx_d.at[bank * nb + b]],
                                     sem_s[b], add=True)
                    start_idx(b, jnp.minimum((grp + 1) * nb + b, NCH - 1),
                              1 - bank)

            def step(t, carry):
                phase(t, t * 2, 0, True)
                phase(t, t * 2 + 1, 1, False)
                return carry
            lax.fori_loop(0, ngrp // 2, step, 0)
            last_bank = 1
            for b in range(nb):
                wait_scatter(b, last_bank)
                wait_idx(b)

        if stage_tab:
            pl.when(c == 0)(lambda: run_edges(su, du, tab_sh))
            pl.when(c == 1)(lambda: run_edges(si, di, tab_sh))
        else:
            pl.when(c == 0)(lambda: run_edges(su, du, tab_u))
            pl.when(c == 1)(lambda: run_edges(si, di, tab_i))
        plsc.subcore_barrier()

        def copy_out(agg_out, cnt_out):
            def out_main():
                pltpu.sync_copy(acc_sh.at[pl.ds(s * SLAB, SLAB)],
                                agg_out.at[pl.ds(s * SLAB, SLAB)])
                if with_counts:
                    pltpu.sync_copy(cnt_sh.at[pl.ds(s * SLAB, SLAB)],
                                    cnt_out.at[pl.ds(s * SLAB, SLAB)])

            def out_tail():
                pltpu.sync_copy(acc_sh.at[pl.ds(15 * SLAB, OTAIL)],
                                agg_out.at[pl.ds(15 * SLAB, OTAIL)])
                if with_counts:
                    pltpu.sync_copy(cnt_sh.at[pl.ds(15 * SLAB, OTAIL)],
                                    cnt_out.at[pl.ds(15 * SLAB, OTAIL)])

            pl.when(s < 15)(out_main)
            pl.when(s == 15)(out_tail)

        if with_counts:
            pl.when(c == 0)(lambda: copy_out(agg_i, cnt_i))
            pl.when(c == 1)(lambda: copy_out(agg_u, cnt_u))
        else:
            pl.when(c == 0)(lambda: copy_out(agg_i, None))
            pl.when(c == 1)(lambda: copy_out(agg_u, None))

    return agg


_agg_l1 = _make_agg_kernel(F, True, NB1)
_agg_l2 = _make_agg_kernel(FO, False, NB, stage_tab=True)


@functools.partial(
    pl.kernel, mesh=_MESH, out_type=[_f32(P_LBL, FO), _f32(P_LBL, FO)],
    compiler_params=_SC_PARAMS,
    scratch_types=[
        pltpu.VMEM((NB, CH), jnp.int32),
        pltpu.VMEM((NB, CH), jnp.int32),
        pltpu.VMEM((NB, CH, FO), jnp.float32),
        pltpu.VMEM((NB, CH, FO), jnp.float32),
    ] + [pltpu.SemaphoreType.DMA] * (2 * NB))
def _pair_gather(zu, zi, l0, l1, f1_out, f2_out, i1, i2, f1, f2, *sems):
    sem_i = sems[:NB]
    sem_g = sems[NB:]
    c = lax.axis_index("c")
    s = lax.axis_index("s")
    wid = s * 2 + c
    base0 = wid * P_SUB

    def start_idx(b, chunk):
        pltpu.async_copy(l0.at[pl.ds(base0 + chunk * CH, CH)],
                         i1.at[b], sem_i[b])
        pltpu.async_copy(l1.at[pl.ds(base0 + chunk * CH, CH)],
                         i2.at[b], sem_i[b])

    def wait_idx(b):
        pltpu.make_async_copy(l0.at[pl.ds(base0, CH)], i1.at[b],
                              sem_i[b]).wait()
        pltpu.make_async_copy(l1.at[pl.ds(base0, CH)], i2.at[b],
                              sem_i[b]).wait()

    for b in range(NB):
        start_idx(b, b)

    def step(t, carry):
        g = t * NB
        handles = []
        for b in range(NB):
            wait_idx(b)
            handles.append(pltpu.async_copy(zu.at[i1.at[b]], f1.at[b],
                                            sem_g[b]))
            handles.append(pltpu.async_copy(zi.at[i2.at[b]], f2.at[b],
                                            sem_g[b]))
        for b in range(NB):
            base = base0 + jnp.minimum(g + b, P_NCH - 1) * CH
            handles[2 * b].wait()
            handles[2 * b + 1].wait()
            pltpu.sync_copy(f1.at[b], f1_out.at[pl.ds(base, CH)])
            pltpu.sync_copy(f2.at[b], f2_out.at[pl.ds(base, CH)])
            start_idx(b, jnp.minimum(g + NB + b, P_NCH - 1))
        return carry
    lax.fori_loop(0, (P_NCH + NB - 1) // NB, step, 0)
    for b in range(NB):
        wait_idx(b)


_DR = 10240


def _dot_body(f1_ref, f2_ref, o_ref):
    o_ref[...] = jnp.sum(f1_ref[...] * f2_ref[...], axis=1).reshape(
        _DR // 128, 128)


def _pair_dot(f1, f2):
    return pl.pallas_call(
        _dot_body,
        grid=(P_LBL // _DR,),
        in_specs=[pl.BlockSpec((_DR, FO), lambda i: (i, 0)),
                  pl.BlockSpec((_DR, FO), lambda i: (i, 0))],
        out_specs=pl.BlockSpec((_DR // 128, 128), lambda i: (i, 0)),
        out_shape=_f32(P_LBL // 128, 128),
    )(f1, f2)


def _dotT(a, b):
    return lax.dot_general(a, b, (((1,), (1,)), ((), ())),
                           preferred_element_type=jnp.float32)


_BR = 1000


def _mlp_one(agg_ref, cnt_ref, x_ref, w1l_ref, b1_ref, w1r_ref,
             w2a_ref, w2b_ref, b2_ref, p_ref, s_ref):
    cnt = jnp.maximum(cnt_ref[:, 0:1], 1.0)
    mean = agg_ref[...] / cnt
    h = jnp.maximum(
        _dotT(mean, w1l_ref[...]) + b1_ref[...] + _dotT(x_ref[...], w1r_ref[...]),
        0.0)
    p_ref[...] = _dotT(h, w2a_ref[...])
    s_ref[...] = _dotT(h, w2b_ref[...]) + b2_ref[...]


def _mlp2_body(*refs):
    _mlp_one(*refs[0:9], *refs[18:20])
    _mlp_one(*refs[9:18], *refs[20:22])


def _mlp2(argsi, argsu):
    full = lambda shp: pl.BlockSpec(shp, lambda i: (0, 0))
    row = lambda w: pl.BlockSpec((_BR, w), lambda i: (i, 0))
    per_type = [row(F), row(CW), row(F), full((F, F)), full((1, F)),
                full((F, F)), full((FO, F)), full((FO, F)), full((1, FO))]

    def prep(a):
        agg, cnt, x, w1l, b1, w1r, w2a, w2b, b2 = a
        return (agg, cnt, x, w1l, b1.reshape(1, F), w1r, w2a, w2b,
                b2.reshape(1, FO))

    return pl.pallas_call(
        _mlp2_body,
        grid=(N // _BR,),
        in_specs=per_type + per_type,
        out_specs=[row(FO)] * 4,
        out_shape=[_f32(N, FO)] * 4,
    )(*prep(argsi), *prep(argsu))


def _finish2_body(agg_i_ref, cnt_i_ref, s_i_ref, agg_u_ref, cnt_u_ref,
                  s_u_ref, z_i_ref, z_u_ref):
    z_i_ref[...] = agg_i_ref[...] / jnp.maximum(cnt_i_ref[:, 0:1], 1.0) + s_i_ref[...]
    z_u_ref[...] = agg_u_ref[...] / jnp.maximum(cnt_u_ref[:, 0:1], 1.0) + s_u_ref[...]


def _finish2(agg2_i, cnt_i, s_i, agg2_u, cnt_u, s_u):
    row = lambda w: pl.BlockSpec((_BR, w), lambda i: (i, 0))
    return pl.pallas_call(
        _finish2_body,
        grid=(N // _BR,),
        in_specs=[row(FO), row(CW), row(FO)] * 2,
        out_specs=[row(FO)] * 2,
        out_shape=[_f32(N, FO)] * 2,
    )(agg2_i, cnt_i, s_i, agg2_u, cnt_u, s_u)


def kernel(x_user, x_item, edge_index_u2i, edge_index_i2u, edge_label_index,
           W1_l_u2i, b1_u2i, W1_r_u2i, W1_l_i2u, b1_i2u, W1_r_i2u,
           W2_l_u2i, b2_u2i, W2_r_u2i, W2_l_i2u, b2_i2u, W2_r_i2u):
    i32 = jnp.int32
    pad_e = E_PAD - E
    su = jnp.concatenate([edge_index_u2i[0].astype(i32), jnp.zeros((pad_e,), i32)])
    du = jnp.concatenate([edge_index_u2i[1].astype(i32), jnp.full((pad_e,), N, i32)])
    si = jnp.concatenate([edge_index_i2u[0].astype(i32), jnp.zeros((pad_e,), i32)])
    di = jnp.concatenate([edge_index_i2u[1].astype(i32), jnp.full((pad_e,), N, i32)])
    pad_l = P_LBL - edge_label_index.shape[1]
    l0 = jnp.concatenate([edge_label_index[0].astype(i32), jnp.zeros((pad_l,), i32)])
    l1 = jnp.concatenate([edge_label_index[1].astype(i32), jnp.zeros((pad_l,), i32)])

    zrow = jnp.zeros((SLAB, F), jnp.float32)
    zcnt = jnp.zeros((SLAB, CW), jnp.float32)
    ones = jnp.ones((CH, CW), jnp.float32)

    agg_i, agg_u, cnt_i, cnt_u = _agg_l1(
        x_user, x_item, su, du, si, di, zrow, zcnt, ones)

    p_item, s_item, p_user, s_user = _mlp2(
        (agg_i, cnt_i, x_item, W1_l_u2i, b1_u2i, W1_r_u2i,
         W2_l_i2u, W2_r_u2i, b2_u2i),
        (agg_u, cnt_u, x_user, W1_l_i2u, b1_i2u, W1_r_i2u,
         W2_l_u2i, W2_r_i2u, b2_i2u))

    zrow2 = jnp.zeros((SLAB, FO), jnp.float32)
    agg2_i, agg2_u = _agg_l2(p_user, p_item, su, du, si, di, zrow2, zcnt, ones)

    z_item, z_user = _finish2(agg2_i, cnt_i, s_item, agg2_u, cnt_u, s_user)

    f1, f2 = _pair_gather(z_user, z_item, l0, l1)
    pred = _pair_dot(f1, f2)
    return pred.reshape(P_LBL)[:E_LBL]


# speedup vs baseline: 1.1807x; 1.0922x over previous
import functools

import jax
import jax.numpy as jnp
from jax import lax
from jax.experimental import pallas as pl
from jax.experimental.pallas import tpu as pltpu
from jax.experimental.pallas import tpu_sc as plsc

N = 10000
F = 128
FO = 64
E = 320000
E_SUB = 20480
E_PAD = E_SUB * 16
CH = 128
NCH = E_SUB // CH
NROW = N + 16
SLAB = 632
ZTAIL = NROW - 15 * SLAB
OTAIL = N - 15 * SLAB
CW = 16

E_LBL = 100000
P_LBL = 102400
P_SUB = P_LBL // 32
P_NCH = P_SUB // CH

_MESH = plsc.VectorSubcoreMesh(core_axis_name="c", subcore_axis_name="s")
_SC_PARAMS = pltpu.CompilerParams(use_tc_tiling_on_sc=False)


def _f32(*shape):
    return jax.ShapeDtypeStruct(shape, jnp.float32)


NB = 4
NB1 = 2


def _make_agg_kernel(width, with_counts, nb, stage_tab=False):
    scratch = [
        pltpu.VMEM((nb, CH), jnp.int32),
        pltpu.VMEM((2 * nb, CH), jnp.int32),
        pltpu.VMEM((nb, CH, width), jnp.float32),
    ] + [pltpu.SemaphoreType.DMA] * (3 * nb) + [
        pltpu.VMEM_SHARED((NROW, width), jnp.float32),
    ]
    if stage_tab:
        scratch.append(pltpu.VMEM_SHARED((N, width), jnp.float32))
    out_type = [_f32(N, width), _f32(N, width)]
    if with_counts:
        scratch.append(pltpu.VMEM((CH, CW), jnp.float32))
        scratch.append(pltpu.VMEM_SHARED((NROW, CW), jnp.float32))
        out_type += [_f32(N, CW), _f32(N, CW)]

    @functools.partial(pl.kernel, mesh=_MESH, out_type=out_type,
                       scratch_types=scratch, compiler_params=_SC_PARAMS)
    def agg(tab_u, tab_i, su, du, si, di, zrow, zcnt, ones_hbm, *rest):
        n_out = 4 if with_counts else 2
        if with_counts:
            agg_i, agg_u, cnt_i, cnt_u = rest[:4]
        else:
            agg_i, agg_u = rest[:2]
            cnt_i = cnt_u = None
        sc = rest[n_out:]
        idx_s, idx_d, rows = sc[:3]
        sem_i = sc[3:3 + nb]
        sem_g = sc[3 + nb:3 + 2 * nb]
        sem_s = sc[3 + 2 * nb:3 + 3 * nb]
        acc_sh = sc[3 + 3 * nb]
        off = 4 + 3 * nb
        if stage_tab:
            tab_sh = sc[off]
            off += 1
        if with_counts:
            ones_v, cnt_sh = sc[off:]
        c = lax.axis_index("c")
        s = lax.axis_index("s")

        def zero_main():
            pltpu.sync_copy(zrow, acc_sh.at[pl.ds(s * SLAB, SLAB)])
            if with_counts:
                pltpu.sync_copy(zcnt, cnt_sh.at[pl.ds(s * SLAB, SLAB)])

        def zero_tail():
            pltpu.sync_copy(zrow.at[pl.ds(0, ZTAIL)],
                            acc_sh.at[pl.ds(15 * SLAB, ZTAIL)])
            if with_counts:
                pltpu.sync_copy(zcnt.at[pl.ds(0, ZTAIL)],
                                cnt_sh.at[pl.ds(15 * SLAB, ZTAIL)])

        pl.when(s < 15)(zero_main)
        pl.when(s == 15)(zero_tail)
        if stage_tab:
            def stage(tab_h):
                pl.when(s < 15)(lambda: pltpu.sync_copy(
                    tab_h.at[pl.ds(s * SLAB, SLAB)],
                    tab_sh.at[pl.ds(s * SLAB, SLAB)]))
                pl.when(s == 15)(lambda: pltpu.sync_copy(
                    tab_h.at[pl.ds(15 * SLAB, OTAIL)],
                    tab_sh.at[pl.ds(15 * SLAB, OTAIL)]))
            pl.when(c == 0)(lambda: stage(tab_u))
            pl.when(c == 1)(lambda: stage(tab_i))
        if with_counts:
            pltpu.sync_copy(ones_hbm, ones_v)
        plsc.subcore_barrier()

        def run_edges(src_h, dst_h, tab_h):
            base0 = s * E_SUB
            ngrp = NCH // nb

            def start_idx(b, chunk, bank):
                pltpu.async_copy(src_h.at[pl.ds(base0 + chunk * CH, CH)],
                                 idx_s.at[b], sem_i[b])
                pltpu.async_copy(dst_h.at[pl.ds(base0 + chunk * CH, CH)],
                                 idx_d.at[bank * nb + b], sem_i[b])

            def wait_idx(b):
                pltpu.make_async_copy(src_h.at[pl.ds(base0, CH)],
                                      idx_s.at[b], sem_i[b]).wait()
                pltpu.make_async_copy(src_h.at[pl.ds(base0, CH)],
                                      idx_s.at[b], sem_i[b]).wait()

            def wait_scatter(b, bank):
                pltpu.make_async_copy(rows.at[b],
                                      acc_sh.at[idx_d.at[bank * nb + b]],
                                      sem_s[b]).wait()

            for b in range(nb):
                start_idx(b, b, 0)

            def phase(t, grp, bank, first):
                handles = []
                for b in range(nb):
                    wait_idx(b)
                    if first:
                        pl.when(t > 0)(
                            lambda b=b: wait_scatter(b, 1 - bank))
                    else:
                        wait_scatter(b, 1 - bank)
                    handles.append(pltpu.async_copy(
                        tab_h.at[idx_s.at[b]], rows.at[b], sem_g[b]))
                    if with_counts:
                        pltpu.sync_copy(ones_v,
                                        cnt_sh.at[idx_d.at[bank * nb + b]],
                                        add=True)
                for b in range(nb):
                    handles[b].wait()
                    pltpu.async_copy(rows.at[b],
                                     acc_sh.at[idx_d.at[bank * nb + b]],
                                     sem_s[b], add=True)
                    start_idx(b, jnp.minimum((grp + 1) * nb + b, NCH - 1),
                              1 - bank)

            def step(t, carry):
                phase(t, t * 2, 0, True)
                phase(t, t * 2 + 1, 1, False)
                return carry
            lax.fori_loop(0, ngrp // 2, step, 0)
            last_bank = 1
            for b in range(nb):
                wait_scatter(b, last_bank)
                wait_idx(b)

        if stage_tab:
            pl.when(c == 0)(lambda: run_edges(su, du, tab_sh))
            pl.when(c == 1)(lambda: run_edges(si, di, tab_sh))
        else:
            pl.when(c == 0)(lambda: run_edges(su, du, tab_u))
            pl.when(c == 1)(lambda: run_edges(si, di, tab_i))
        plsc.subcore_barrier()

        def copy_out(agg_out, cnt_out):
            def out_main():
                pltpu.sync_copy(acc_sh.at[pl.ds(s * SLAB, SLAB)],
                                agg_out.at[pl.ds(s * SLAB, SLAB)])
                if with_counts:
                    pltpu.sync_copy(cnt_sh.at[pl.ds(s * SLAB, SLAB)],
                                    cnt_out.at[pl.ds(s * SLAB, SLAB)])

            def out_tail():
                pltpu.sync_copy(acc_sh.at[pl.ds(15 * SLAB, OTAIL)],
                                agg_out.at[pl.ds(15 * SLAB, OTAIL)])
                if with_counts:
                    pltpu.sync_copy(cnt_sh.at[pl.ds(15 * SLAB, OTAIL)],
                                    cnt_out.at[pl.ds(15 * SLAB, OTAIL)])

            pl.when(s < 15)(out_main)
            pl.when(s == 15)(out_tail)

        if with_counts:
            pl.when(c == 0)(lambda: copy_out(agg_i, cnt_i))
            pl.when(c == 1)(lambda: copy_out(agg_u, cnt_u))
        else:
            pl.when(c == 0)(lambda: copy_out(agg_i, None))
            pl.when(c == 1)(lambda: copy_out(agg_u, None))

    return agg


_agg_l1 = _make_agg_kernel(F, True, NB1)
_agg_l2 = _make_agg_kernel(FO, False, NB, stage_tab=True)


NBP = 2


@functools.partial(
    pl.kernel, mesh=_MESH, out_type=[_f32(P_LBL, FO), _f32(P_LBL, FO)],
    compiler_params=_SC_PARAMS,
    scratch_types=[
        pltpu.VMEM((NBP, CH), jnp.int32),
        pltpu.VMEM((NBP, CH), jnp.int32),
        pltpu.VMEM((NBP, CH, FO), jnp.float32),
        pltpu.VMEM((NBP, CH, FO), jnp.float32),
        pltpu.VMEM_SHARED((N, FO), jnp.float32),
        pltpu.VMEM_SHARED((N, FO), jnp.float32),
    ] + [pltpu.SemaphoreType.DMA] * (2 * NBP))
def _pair_gather(zu, zi, l0, l1, f1_out, f2_out, i1, i2, f1, f2,
                 zu_sh, zi_sh, *sems):
    sem_i = sems[:NBP]
    sem_g = sems[NBP:]
    c = lax.axis_index("c")
    s = lax.axis_index("s")
    wid = s * 2 + c
    base0 = wid * P_SUB

    def stage(src_h, dst_sh):
        pl.when(s < 15)(lambda: pltpu.sync_copy(
            src_h.at[pl.ds(s * SLAB, SLAB)], dst_sh.at[pl.ds(s * SLAB, SLAB)]))
        pl.when(s == 15)(lambda: pltpu.sync_copy(
            src_h.at[pl.ds(15 * SLAB, OTAIL)],
            dst_sh.at[pl.ds(15 * SLAB, OTAIL)]))
    stage(zu, zu_sh)
    stage(zi, zi_sh)
    plsc.subcore_barrier()

    def start_idx(b, chunk):
        pltpu.async_copy(l0.at[pl.ds(base0 + chunk * CH, CH)],
                         i1.at[b], sem_i[b])
        pltpu.async_copy(l1.at[pl.ds(base0 + chunk * CH, CH)],
                         i2.at[b], sem_i[b])

    def wait_idx(b):
        pltpu.make_async_copy(l0.at[pl.ds(base0, CH)], i1.at[b],
                              sem_i[b]).wait()
        pltpu.make_async_copy(l1.at[pl.ds(base0, CH)], i2.at[b],
                              sem_i[b]).wait()

    for b in range(NBP):
        start_idx(b, b)

    def step(t, carry):
        g = t * NBP
        handles = []
        for b in range(NBP):
            wait_idx(b)
            handles.append(pltpu.async_copy(zu_sh.at[i1.at[b]], f1.at[b],
                                            sem_g[b]))
            handles.append(pltpu.async_copy(zi_sh.at[i2.at[b]], f2.at[b],
                                            sem_g[b]))
        for b in range(NBP):
            base = base0 + jnp.minimum(g + b, P_NCH - 1) * CH
            handles[2 * b].wait()
            handles[2 * b + 1].wait()
            pltpu.sync_copy(f1.at[b], f1_out.at[pl.ds(base, CH)])
            pltpu.sync_copy(f2.at[b], f2_out.at[pl.ds(base, CH)])
            start_idx(b, jnp.minimum(g + NBP + b, P_NCH - 1))
        return carry
    lax.fori_loop(0, (P_NCH + NBP - 1) // NBP, step, 0)
    for b in range(NBP):
        wait_idx(b)


_DR = 10240


def _dot_body(f1_ref, f2_ref, o_ref):
    o_ref[...] = jnp.sum(f1_ref[...] * f2_ref[...], axis=1).reshape(
        _DR // 128, 128)


def _pair_dot(f1, f2):
    return pl.pallas_call(
        _dot_body,
        grid=(P_LBL // _DR,),
        in_specs=[pl.BlockSpec((_DR, FO), lambda i: (i, 0)),
                  pl.BlockSpec((_DR, FO), lambda i: (i, 0))],
        out_specs=pl.BlockSpec((_DR // 128, 128), lambda i: (i, 0)),
        out_shape=_f32(P_LBL // 128, 128),
    )(f1, f2)


def _dotT(a, b):
    return lax.dot_general(a, b, (((1,), (1,)), ((), ())),
                           preferred_element_type=jnp.float32)


_BR = 1000


def _mlp_one(agg_ref, cnt_ref, x_ref, w1l_ref, b1_ref, w1r_ref,
             w2a_ref, w2b_ref, b2_ref, p_ref, s_ref):
    cnt = jnp.maximum(cnt_ref[:, 0:1], 1.0)
    mean = agg_ref[...] / cnt
    h = jnp.maximum(
        _dotT(mean, w1l_ref[...]) + b1_ref[...] + _dotT(x_ref[...], w1r_ref[...]),
        0.0)
    p_ref[...] = _dotT(h, w2a_ref[...])
    s_ref[...] = _dotT(h, w2b_ref[...]) + b2_ref[...]


def _mlp2_body(*refs):
    _mlp_one(*refs[0:9], *refs[18:20])
    _mlp_one(*refs[9:18], *refs[20:22])


def _mlp2(argsi, argsu):
    full = lambda shp: pl.BlockSpec(shp, lambda i: (0, 0))
    row = lambda w: pl.BlockSpec((_BR, w), lambda i: (i, 0))
    per_type = [row(F), row(CW), row(F), full((F, F)), full((1, F)),
                full((F, F)), full((FO, F)), full((FO, F)), full((1, FO))]

    def prep(a):
        agg, cnt, x, w1l, b1, w1r, w2a, w2b, b2 = a
        return (agg, cnt, x, w1l, b1.reshape(1, F), w1r, w2a, w2b,
                b2.reshape(1, FO))

    return pl.pallas_call(
        _mlp2_body,
        grid=(N // _BR,),
        in_specs=per_type + per_type,
        out_specs=[row(FO)] * 4,
        out_shape=[_f32(N, FO)] * 4,
    )(*prep(argsi), *prep(argsu))


def _finish2_body(agg_i_ref, cnt_i_ref, s_i_ref, agg_u_ref, cnt_u_ref,
                  s_u_ref, z_i_ref, z_u_ref):
    z_i_ref[...] = agg_i_ref[...] / jnp.maximum(cnt_i_ref[:, 0:1], 1.0) + s_i_ref[...]
    z_u_ref[...] = agg_u_ref[...] / jnp.maximum(cnt_u_ref[:, 0:1], 1.0) + s_u_ref[...]


def _finish2(agg2_i, cnt_i, s_i, agg2_u, cnt_u, s_u):
    row = lambda w: pl.BlockSpec((_BR, w), lambda i: (i, 0))
    return pl.pallas_call(
        _finish2_body,
        grid=(N // _BR,),
        in_specs=[row(FO), row(CW), row(FO)] * 2,
        out_specs=[row(FO)] * 2,
        out_shape=[_f32(N, FO)] * 2,
    )(agg2_i, cnt_i, s_i, agg2_u, cnt_u, s_u)


def kernel(x_user, x_item, edge_index_u2i, edge_index_i2u, edge_label_index,
           W1_l_u2i, b1_u2i, W1_r_u2i, W1_l_i2u, b1_i2u, W1_r_i2u,
           W2_l_u2i, b2_u2i, W2_r_u2i, W2_l_i2u, b2_i2u, W2_r_i2u):
    i32 = jnp.int32
    pad_e = E_PAD - E
    su = jnp.concatenate([edge_index_u2i[0].astype(i32), jnp.zeros((pad_e,), i32)])
    du = jnp.concatenate([edge_index_u2i[1].astype(i32), jnp.full((pad_e,), N, i32)])
    si = jnp.concatenate([edge_index_i2u[0].astype(i32), jnp.zeros((pad_e,), i32)])
    di = jnp.concatenate([edge_index_i2u[1].astype(i32), jnp.full((pad_e,), N, i32)])
    pad_l = P_LBL - edge_label_index.shape[1]
    l0 = jnp.concatenate([edge_label_index[0].astype(i32), jnp.zeros((pad_l,), i32)])
    l1 = jnp.concatenate([edge_label_index[1].astype(i32), jnp.zeros((pad_l,), i32)])

    zrow = jnp.zeros((SLAB, F), jnp.float32)
    zcnt = jnp.zeros((SLAB, CW), jnp.float32)
    ones = jnp.ones((CH, CW), jnp.float32)

    agg_i, agg_u, cnt_i, cnt_u = _agg_l1(
        x_user, x_item, su, du, si, di, zrow, zcnt, ones)

    p_item, s_item, p_user, s_user = _mlp2(
        (agg_i, cnt_i, x_item, W1_l_u2i, b1_u2i, W1_r_u2i,
         W2_l_i2u, W2_r_u2i, b2_u2i),
        (agg_u, cnt_u, x_user, W1_l_i2u, b1_i2u, W1_r_i2u,
         W2_l_u2i, W2_r_i2u, b2_i2u))

    zrow2 = jnp.zeros((SLAB, FO), jnp.float32)
    agg2_i, agg2_u = _agg_l2(p_user, p_item, su, du, si, di, zrow2, zcnt, ones)

    z_item, z_user = _finish2(agg2_i, cnt_i, s_item, agg2_u, cnt_u, s_user)

    f1, f2 = _pair_gather(z_user, z_item, l0, l1)
    pred = _pair_dot(f1, f2)
    return pred.reshape(P_LBL)[:E_LBL]
